# baseline (speedup 1.0000x reference)
import functools

import jax
import jax.numpy as jnp
from jax import lax
from jax.experimental import pallas as pl
from jax.experimental.pallas import tpu as pltpu
from jax.experimental.pallas import tpu_sc as plsc

RADIUS2 = 0.1 * 0.1
NS = 32
NWORKERS = 32
LANES = 16


def _mask_pack_body(xi_ref, xyzT_ref, out_ref, *, MA):
    N = xyzT_ref.shape[2]
    xi = xi_ref[0]
    xjT = xyzT_ref[0]

    d0 = xi[:, 0].reshape(MA, 1) - xjT[0:1, :]
    d1 = xi[:, 1].reshape(MA, 1) - xjT[1:2, :]
    d2c = xi[:, 2].reshape(MA, 1) - xjT[2:3, :]
    d2 = (d0 * d0 + d1 * d1) + d2c * d2c
    winf = (d2 < RADIUS2).astype(jnp.float32)

    rj = lax.broadcasted_iota(jnp.int32, (128, 8), 0)
    cw = lax.broadcasted_iota(jnp.int32, (128, 8), 1)
    pmat = jnp.where((rj >> 4) == cw, 1 << (rj & 15), 0).astype(jnp.float32)

    for c in range(N // 128):
        words8 = lax.dot_general(winf[:, c * 128:(c + 1) * 128], pmat,
                                 (((1,), (0,)), ((), ())),
                                 preferred_element_type=jnp.float32)
        out_ref[:, c * 8:(c + 1) * 8] = words8.astype(jnp.int32)


def _sc_select_body(words_hbm, xyzT_hbm, relout_hbm, dens_hbm,
                    wbuf, xv, yv, zv, nzv, nzb, selidx, staging, densbuf,
                    *, N, CP):
    total = dens_hbm.shape[0]
    per_w = total // NWORKERS
    nchunk = per_w // CP
    cid = lax.axis_index("c")
    sid = lax.axis_index("s")
    wid = sid * 2 + cid
    base = wid * per_w
    b = base // N

    pltpu.sync_copy(xyzT_hbm.at[pl.ds((b * 3 + 0) * N, N)], xv)
    pltpu.sync_copy(xyzT_hbm.at[pl.ds((b * 3 + 1) * N, N)], yv)
    pltpu.sync_copy(xyzT_hbm.at[pl.ds((b * 3 + 2) * N, N)], zv)

    iota = lax.iota(jnp.int32, LANES)
    zf = jnp.zeros((LANES,), jnp.float32)

    def zero_body(i, carry):
        for q in range(128 // LANES):
            staging[i, pl.ds(q * LANES, LANES)] = zf
        return carry
    lax.fori_loop(0, CP, zero_body, 0)

    def chunk_body(ch, carry):
        g0 = base + ch * CP
        pltpu.sync_copy(words_hbm.at[pl.ds(g0, CP)], wbuf)

        def point_body(pi, carry2):
            p_loc = g0 + pi - b * N
            fill = jnp.full((LANES,), p_loc, jnp.int32)
            selidx[pl.ds(0, LANES)] = fill
            selidx[pl.ds(16, LANES)] = fill

            offv = jnp.zeros((LANES,), jnp.int32)
            for w in range(16):
                v = wbuf[pi, pl.ds(w * 16, LANES)]
                m = v != 0
                mi = m.astype(jnp.int32)
                pos = offv - 1 + plsc.cumsum(mi)
                enc = v | ((jnp.full((LANES,), w * 16, jnp.int32) + iota) << 16)
                plsc.store_scatter(nzv, [pos], enc, mask=m)
                offv = offv + plsc.all_reduce_population_count(m)
            off = jnp.max(offv)

            def one_word(k_idx, valid, nselv, cntv):
                kf = jnp.full((LANES,), k_idx, jnp.int32)
                ev = plsc.load_gather(nzv, [kf])
                wv = ev & 0xFFFF
                wb = ev >> 16
                bits = jnp.logical_and(((wv >> iota) & 1) == 1, valid)
                bi = bits.astype(jnp.int32)
                pcv = plsc.all_reduce_population_count(bits)
                slots = jnp.minimum(nselv - 1 + plsc.cumsum(bi), 47)
                plsc.store_scatter(selidx, [slots], wb * 16 + iota, mask=bits)
                return nselv + pcv, cntv + pcv

            tru = iota < 16
            def word_body(k, carry3):
                nselv, cntv = carry3
                nselv, cntv = one_word(2 * k, tru, nselv, cntv)
                valid2 = jnp.full((LANES,), 2 * k + 1, jnp.int32) < offv
                nselv, cntv = one_word(2 * k + 1, valid2, nselv, cntv)
                return nselv, cntv

            zi = jnp.zeros((LANES,), jnp.int32)
            _, cntv = lax.fori_loop(0, (off + 1) // 2, word_body, (zi, zi))

            pf = jnp.full((LANES,), p_loc, jnp.int32)
            px = plsc.load_gather(xv, [pf])
            py = plsc.load_gather(yv, [pf])
            pz = plsc.load_gather(zv, [pf])
            for s in range(2):
                idxv = selidx[pl.ds(s * LANES, LANES)]
                for coord, (buf, pc) in enumerate(((xv, px), (yv, py), (zv, pz))):
                    g = plsc.load_gather(buf, [idxv])
                    rel = g - pc
                    pos = iota * 4 + jnp.full((LANES,), 64 * s + coord, jnp.int32)
                    plsc.store_scatter(staging,
                                       [jnp.full((LANES,), pi, jnp.int32), pos],
                                       rel)

            densv = jnp.minimum(cntv, NS).astype(jnp.float32) * (1.0 / NS)
            plsc.store_scatter(densbuf, [jnp.full((LANES,), pi, jnp.int32)],
                               densv, mask=iota == 0)
            return carry2

        lax.fori_loop(0, CP, point_body, 0)
        pltpu.sync_copy(staging, relout_hbm.at[pl.ds(g0, CP)])
        pltpu.sync_copy(densbuf, dens_hbm.at[pl.ds(g0, CP)])
        return carry

    lax.fori_loop(0, nchunk, chunk_body, 0)


def _mlp_pool_body(rel_ref, dens_ref, w1e_ref, b1e_ref, w2e_ref, b2e_ref,
                   out_ref):
    rel = rel_ref[...]
    h1 = jax.nn.relu(
        lax.dot_general(rel, w1e_ref[...], (((1,), (0,)), ((), ())),
                        preferred_element_type=jnp.float32) + b1e_ref[...])
    h2 = jax.nn.relu(
        lax.dot_general(h1, w2e_ref[...], (((1,), (0,)), ((), ())),
                        preferred_element_type=jnp.float32) + b2e_ref[...])
    m = h2
    for half in (512, 256, 128, 64, 32):
        m = jnp.maximum(m[:, :half], m[:, half:])
    out_ref[:, 0:32] = m
    out_ref[:, 32:33] = dens_ref[...]


def kernel(xyz, W1, b1, W2, b2):
    B, N, _ = xyz.shape
    xyzT = jnp.transpose(xyz, (0, 2, 1))

    w1pad = jnp.concatenate([W1, jnp.zeros((32, 1), jnp.float32)], axis=1)
    eye32 = jnp.eye(32, dtype=jnp.float32)
    w1e = jnp.kron(eye32, w1pad.T)
    w2e = jnp.kron(eye32, W2.T)
    b1e = jnp.tile(b1, NS)
    b2e = jnp.tile(b2, NS)

    MA = 512
    CP = 32
    PC = 512
    mesh = plsc.VectorSubcoreMesh(core_axis_name="c", subcore_axis_name="s")
    sc_select = pl.kernel(
        functools.partial(_sc_select_body, N=N, CP=CP),
        out_type=[jax.ShapeDtypeStruct((N, 128), jnp.float32),
                  jax.ShapeDtypeStruct((N,), jnp.float32)],
        mesh=mesh,
        compiler_params=pltpu.CompilerParams(needs_layout_passes=False),
        scratch_types=[
            pltpu.VMEM((CP, 256), jnp.int32),
            pltpu.VMEM((N,), jnp.float32),
            pltpu.VMEM((N,), jnp.float32),
            pltpu.VMEM((N,), jnp.float32),
            pltpu.VMEM((272,), jnp.int32),
            pltpu.VMEM((272,), jnp.int32),
            pltpu.VMEM((48,), jnp.int32),
            pltpu.VMEM((CP, 128), jnp.float32),
            pltpu.VMEM((CP,), jnp.float32),
        ],
    )

    outs = []
    for bi_ in range(B):
        words_b = pl.pallas_call(
            functools.partial(_mask_pack_body, MA=MA),
            grid=(N // MA,),
            in_specs=[
                pl.BlockSpec((1, MA, 3), lambda m_, b_=bi_: (b_, m_, 0)),
                pl.BlockSpec((1, 3, N), lambda m_, b_=bi_: (b_, 0, 0)),
            ],
            out_specs=pl.BlockSpec((MA, 256), lambda m_: (m_, 0)),
            out_shape=jax.ShapeDtypeStruct((N, 256), jnp.int32),
        )(xyz, xyzT)

        rel_b, dens_b = sc_select(words_b, xyzT[bi_].reshape(3 * N))

        out_b = pl.pallas_call(
            _mlp_pool_body,
            grid=(N // PC,),
            in_specs=[
                pl.BlockSpec((PC, 128), lambda i: (i, 0)),
                pl.BlockSpec((PC, 1), lambda i: (i, 0)),
                pl.BlockSpec((128, 1024), lambda i: (0, 0)),
                pl.BlockSpec((1024,), lambda i: (0,)),
                pl.BlockSpec((1024, 1024), lambda i: (0, 0)),
                pl.BlockSpec((1024,), lambda i: (0,)),
            ],
            out_specs=pl.BlockSpec((PC, 33), lambda i: (i, 0)),
            out_shape=jax.ShapeDtypeStruct((N, 33), jnp.float32),
        )(rel_b, dens_b.reshape(N, 1), w1e, b1e, w2e, b2e)
        outs.append(out_b)

    return jnp.stack(outs, axis=0)


# speedup vs baseline: 1.0461x; 1.0461x over previous
import functools

import jax
import jax.numpy as jnp
from jax import lax
from jax.experimental import pallas as pl
from jax.experimental.pallas import tpu as pltpu
from jax.experimental.pallas import tpu_sc as plsc

RADIUS2 = 0.1 * 0.1
NS = 32
NWORKERS = 32
LANES = 16


def _mask_pack_body(xi_ref, xyzT_ref, out_ref, *, MA):
    N = xyzT_ref.shape[2]
    xi = xi_ref[0]
    xjT = xyzT_ref[0]

    d0 = xi[:, 0].reshape(MA, 1) - xjT[0:1, :]
    d1 = xi[:, 1].reshape(MA, 1) - xjT[1:2, :]
    d2c = xi[:, 2].reshape(MA, 1) - xjT[2:3, :]
    d2 = (d0 * d0 + d1 * d1) + d2c * d2c
    winf = (d2 < RADIUS2).astype(jnp.float32)

    rj = lax.broadcasted_iota(jnp.int32, (128, 8), 0)
    cw = lax.broadcasted_iota(jnp.int32, (128, 8), 1)
    pmat = jnp.where((rj >> 4) == cw, 1 << (rj & 15), 0).astype(jnp.float32)

    for c in range(N // 128):
        words8 = lax.dot_general(winf[:, c * 128:(c + 1) * 128], pmat,
                                 (((1,), (0,)), ((), ())),
                                 preferred_element_type=jnp.float32)
        out_ref[:, c * 8:(c + 1) * 8] = words8.astype(jnp.int32)


def _sc_select_body(words_hbm, xyzT_hbm, relout_hbm, dens_hbm,
                    wbuf, xv, yv, zv, nzv, nzb, selidx, staging, densbuf,
                    *, N, CP):
    total = dens_hbm.shape[0]
    per_w = total // NWORKERS
    nchunk = per_w // CP
    cid = lax.axis_index("c")
    sid = lax.axis_index("s")
    wid = sid * 2 + cid
    base = wid * per_w
    b = base // N

    pltpu.sync_copy(xyzT_hbm.at[pl.ds((b * 3 + 0) * N, N)], xv)
    pltpu.sync_copy(xyzT_hbm.at[pl.ds((b * 3 + 1) * N, N)], yv)
    pltpu.sync_copy(xyzT_hbm.at[pl.ds((b * 3 + 2) * N, N)], zv)

    iota = lax.iota(jnp.int32, LANES)
    zf = jnp.zeros((LANES,), jnp.float32)

    def zero_body(i, carry):
        for q in range(128 // LANES):
            staging[i, pl.ds(q * LANES, LANES)] = zf
        return carry
    lax.fori_loop(0, CP, zero_body, 0)

    def chunk_body(ch, carry):
        g0 = base + ch * CP
        pltpu.sync_copy(words_hbm.at[pl.ds(g0, CP)], wbuf)

        def point_body(pi, carry2):
            p_loc = g0 + pi - b * N
            fill = jnp.full((LANES,), p_loc, jnp.int32)
            selidx[pl.ds(0, LANES)] = fill
            selidx[pl.ds(16, LANES)] = fill

            offv = jnp.zeros((LANES,), jnp.int32)
            for w in range(16):
                v = wbuf[pi, pl.ds(w * 16, LANES)]
                m = v != 0
                mi = m.astype(jnp.int32)
                pos = offv - 1 + plsc.cumsum(mi)
                plsc.store_scatter(nzv, [pos], v, mask=m)
                plsc.store_scatter(nzb, [pos],
                                   jnp.full((LANES,), w * 16, jnp.int32) + iota,
                                   mask=m)
                offv = offv + plsc.all_reduce_population_count(m)
            off = jnp.max(offv)

            def word_body(k, carry3):
                nselv, cntv = carry3
                kf = jnp.full((LANES,), k, jnp.int32)
                wv = plsc.load_gather(nzv, [kf])
                wb = plsc.load_gather(nzb, [kf])
                bits = ((wv >> iota) & 1) == 1
                bi = bits.astype(jnp.int32)
                pcv = plsc.all_reduce_population_count(bits)
                slots = jnp.minimum(nselv - 1 + plsc.cumsum(bi), 47)
                plsc.store_scatter(selidx, [slots], wb * 16 + iota, mask=bits)
                return nselv + pcv, cntv + pcv

            zi = jnp.zeros((LANES,), jnp.int32)
            _, cntv = lax.fori_loop(0, off, word_body, (zi, zi))

            pf = jnp.full((LANES,), p_loc, jnp.int32)
            px = plsc.load_gather(xv, [pf])
            py = plsc.load_gather(yv, [pf])
            pz = plsc.load_gather(zv, [pf])
            for s in range(2):
                idxv = selidx[pl.ds(s * LANES, LANES)]
                for coord, (buf, pc) in enumerate(((xv, px), (yv, py), (zv, pz))):
                    g = plsc.load_gather(buf, [idxv])
                    rel = g - pc
                    pos = iota * 4 + jnp.full((LANES,), 64 * s + coord, jnp.int32)
                    plsc.store_scatter(staging,
                                       [jnp.full((LANES,), pi, jnp.int32), pos],
                                       rel)

            densv = jnp.minimum(cntv, NS).astype(jnp.float32) * (1.0 / NS)
            plsc.store_scatter(densbuf, [jnp.full((LANES,), pi, jnp.int32)],
                               densv, mask=iota == 0)
            return carry2

        lax.fori_loop(0, CP, point_body, 0)
        pltpu.sync_copy(staging, relout_hbm.at[pl.ds(g0, CP)])
        pltpu.sync_copy(densbuf, dens_hbm.at[pl.ds(g0, CP)])
        return carry

    lax.fori_loop(0, nchunk, chunk_body, 0)


def _mlp_pool_body(rel_ref, dens_ref, w1e_ref, b1e_ref, w2e_ref, b2e_ref,
                   out_ref):
    rel = rel_ref[...]
    h1 = jax.nn.relu(
        lax.dot_general(rel, w1e_ref[...], (((1,), (0,)), ((), ())),
                        preferred_element_type=jnp.float32) + b1e_ref[...])
    h2 = jax.nn.relu(
        lax.dot_general(h1, w2e_ref[...], (((1,), (0,)), ((), ())),
                        preferred_element_type=jnp.float32) + b2e_ref[...])
    m = h2
    for half in (512, 256, 128, 64, 32):
        m = jnp.maximum(m[:, :half], m[:, half:])
    out_ref[:, 0:32] = m
    out_ref[:, 32:33] = dens_ref[...]


def kernel(xyz, W1, b1, W2, b2):
    B, N, _ = xyz.shape
    xyzT = jnp.transpose(xyz, (0, 2, 1))

    w1pad = jnp.concatenate([W1, jnp.zeros((32, 1), jnp.float32)], axis=1)
    eye32 = jnp.eye(32, dtype=jnp.float32)
    w1e = jnp.kron(eye32, w1pad.T)
    w2e = jnp.kron(eye32, W2.T)
    b1e = jnp.tile(b1, NS)
    b2e = jnp.tile(b2, NS)

    MA = 512
    CP = 128
    PC = 512
    mesh = plsc.VectorSubcoreMesh(core_axis_name="c", subcore_axis_name="s")
    sc_select = pl.kernel(
        functools.partial(_sc_select_body, N=N, CP=CP),
        out_type=[jax.ShapeDtypeStruct((N, 128), jnp.float32),
                  jax.ShapeDtypeStruct((N,), jnp.float32)],
        mesh=mesh,
        compiler_params=pltpu.CompilerParams(needs_layout_passes=False),
        scratch_types=[
            pltpu.VMEM((CP, 256), jnp.int32),
            pltpu.VMEM((N,), jnp.float32),
            pltpu.VMEM((N,), jnp.float32),
            pltpu.VMEM((N,), jnp.float32),
            pltpu.VMEM((272,), jnp.int32),
            pltpu.VMEM((272,), jnp.int32),
            pltpu.VMEM((48,), jnp.int32),
            pltpu.VMEM((CP, 128), jnp.float32),
            pltpu.VMEM((CP,), jnp.float32),
        ],
    )

    outs = []
    for bi_ in range(B):
        words_b = pl.pallas_call(
            functools.partial(_mask_pack_body, MA=MA),
            grid=(N // MA,),
            in_specs=[
                pl.BlockSpec((1, MA, 3), lambda m_, b_=bi_: (b_, m_, 0)),
                pl.BlockSpec((1, 3, N), lambda m_, b_=bi_: (b_, 0, 0)),
            ],
            out_specs=pl.BlockSpec((MA, 256), lambda m_: (m_, 0)),
            out_shape=jax.ShapeDtypeStruct((N, 256), jnp.int32),
        )(xyz, xyzT)

        rel_b, dens_b = sc_select(words_b, xyzT[bi_].reshape(3 * N))

        out_b = pl.pallas_call(
            _mlp_pool_body,
            grid=(N // PC,),
            in_specs=[
                pl.BlockSpec((PC, 128), lambda i: (i, 0)),
                pl.BlockSpec((PC, 1), lambda i: (i, 0)),
                pl.BlockSpec((128, 1024), lambda i: (0, 0)),
                pl.BlockSpec((1024,), lambda i: (0,)),
                pl.BlockSpec((1024, 1024), lambda i: (0, 0)),
                pl.BlockSpec((1024,), lambda i: (0,)),
            ],
            out_specs=pl.BlockSpec((PC, 33), lambda i: (i, 0)),
            out_shape=jax.ShapeDtypeStruct((N, 33), jnp.float32),
        )(rel_b, dens_b.reshape(N, 1), w1e, b1e, w2e, b2e)
        outs.append(out_b)

    return jnp.stack(outs, axis=0)
